# baseline (speedup 1.0000x reference)
import jax
import jax.numpy as jnp
from jax.experimental import pallas as pl
from jax.experimental.pallas import tpu as pltpu

D_MODEL = 768
N_PAD = 64
BLOCK_M = 4096


def _fused_heads_kernel(x0_ref, x1_ref, w_ref, b_ref, ow_ref, oh_ref, od_ref, ol_ref):
    res = (
        jnp.dot(x0_ref[...], w_ref[0:384, :], preferred_element_type=jnp.float32)
        + jnp.dot(x1_ref[...], w_ref[384:768, :], preferred_element_type=jnp.float32)
        + b_ref[...]
    )
    ow_ref[...] = res[:, 0:16]
    oh_ref[...] = res[:, 16:32]
    od_ref[...] = res[:, 32:47]
    ol_ref[...] = res[:, 47:63]


def kernel(conditioning, W_w, b_w, W_h, b_h, W_d, b_d, W_l, b_l):
    B, T, D = conditioning.shape
    M = B * T
    x = conditioning.reshape(M, D)

    W_cat = jnp.concatenate([W_w, W_h, W_d, W_l], axis=0)
    W_cat = jnp.pad(W_cat, ((0, N_PAD - W_cat.shape[0]), (0, 0))).T
    b_cat = jnp.concatenate([b_w, b_h, b_d, b_l], axis=0)
    b_cat = jnp.pad(b_cat, (0, N_PAD - b_cat.shape[0])).reshape(1, N_PAD)

    grid = (M // BLOCK_M,)
    ow, oh, od, ol = pl.pallas_call(
        _fused_heads_kernel,
        grid=grid,
        in_specs=[
            pl.BlockSpec((BLOCK_M, D // 2), lambda i: (i, 0)),
            pl.BlockSpec((BLOCK_M, D // 2), lambda i: (i, 1)),
            pl.BlockSpec((D, N_PAD), lambda i: (0, 0)),
            pl.BlockSpec((1, N_PAD), lambda i: (0, 0)),
        ],
        out_specs=[
            pl.BlockSpec((BLOCK_M, 16), lambda i: (i, 0)),
            pl.BlockSpec((BLOCK_M, 16), lambda i: (i, 0)),
            pl.BlockSpec((BLOCK_M, 15), lambda i: (i, 0)),
            pl.BlockSpec((BLOCK_M, 16), lambda i: (i, 0)),
        ],
        out_shape=[
            jax.ShapeDtypeStruct((M, 16), jnp.float32),
            jax.ShapeDtypeStruct((M, 16), jnp.float32),
            jax.ShapeDtypeStruct((M, 15), jnp.float32),
            jax.ShapeDtypeStruct((M, 16), jnp.float32),
        ],
        compiler_params=pltpu.CompilerParams(
            dimension_semantics=("parallel",),
        ),
    )(x, x, W_cat, b_cat)

    return (
        ow.reshape(B, T, 16),
        oh.reshape(B, T, 16),
        od.reshape(B, T, 15),
        ol.reshape(B, T, 16),
    )


# speedup vs baseline: 1.0056x; 1.0056x over previous
import jax
import jax.numpy as jnp
from jax.experimental import pallas as pl
from jax.experimental.pallas import tpu as pltpu

D_MODEL = 768
N_PAD = 64
BLOCK_T = 4096


def _fused_heads_kernel(x_ref, w_ref, b_ref, ow_ref, oh_ref, od_ref, ol_ref):
    res = (
        jnp.dot(x_ref[0], w_ref[...], preferred_element_type=jnp.float32)
        + b_ref[...]
    )
    ow_ref[0] = res[:, 0:16]
    oh_ref[0] = res[:, 16:32]
    od_ref[0] = res[:, 32:47]
    ol_ref[0] = res[:, 47:63]


def kernel(conditioning, W_w, b_w, W_h, b_h, W_d, b_d, W_l, b_l):
    B, T, D = conditioning.shape

    W_cat = jnp.concatenate([W_w, W_h, W_d, W_l], axis=0)
    W_cat = jnp.pad(W_cat, ((0, N_PAD - W_cat.shape[0]), (0, 0))).T
    b_cat = jnp.concatenate([b_w, b_h, b_d, b_l], axis=0)
    b_cat = jnp.pad(b_cat, (0, N_PAD - b_cat.shape[0])).reshape(1, N_PAD)

    grid = (B, T // BLOCK_T)
    ow, oh, od, ol = pl.pallas_call(
        _fused_heads_kernel,
        grid=grid,
        in_specs=[
            pl.BlockSpec((1, BLOCK_T, D), lambda b, j: (b, j, 0)),
            pl.BlockSpec((D, N_PAD), lambda b, j: (0, 0)),
            pl.BlockSpec((1, N_PAD), lambda b, j: (0, 0)),
        ],
        out_specs=[
            pl.BlockSpec((1, BLOCK_T, 16), lambda b, j: (b, j, 0)),
            pl.BlockSpec((1, BLOCK_T, 16), lambda b, j: (b, j, 0)),
            pl.BlockSpec((1, BLOCK_T, 15), lambda b, j: (b, j, 0)),
            pl.BlockSpec((1, BLOCK_T, 16), lambda b, j: (b, j, 0)),
        ],
        out_shape=[
            jax.ShapeDtypeStruct((B, T, 16), jnp.float32),
            jax.ShapeDtypeStruct((B, T, 16), jnp.float32),
            jax.ShapeDtypeStruct((B, T, 15), jnp.float32),
            jax.ShapeDtypeStruct((B, T, 16), jnp.float32),
        ],
        compiler_params=pltpu.CompilerParams(
            dimension_semantics=("parallel", "parallel"),
        ),
    )(conditioning, W_cat, b_cat)

    return (ow, oh, od, ol)


# speedup vs baseline: 2.1159x; 2.1042x over previous
import jax
import jax.numpy as jnp
from jax.experimental import pallas as pl
from jax.experimental.pallas import tpu as pltpu

D_MODEL = 768
N_PAD = 64
BLOCK_T = 4096


def _fused_heads_kernel(x_ref, w_ref, b_ref, ow_ref, oh_ref, ol_ref, od_ref):
    res = (
        jax.lax.dot_general(
            w_ref[...],
            x_ref[0],
            dimension_numbers=(((1,), (1,)), ((), ())),
            preferred_element_type=jnp.float32,
        )
        + b_ref[...]
    )
    ow_ref[0] = res[0:16, :]
    oh_ref[0] = res[16:32, :]
    ol_ref[0] = res[32:48, :]
    od_ref[0] = res[48:63, :]


def kernel(conditioning, W_w, b_w, W_h, b_h, W_d, b_d, W_l, b_l):
    B, T, D = conditioning.shape

    W_cat = jnp.concatenate(
        [W_w, W_h, W_l, W_d, jnp.zeros((1, D), jnp.float32)], axis=0
    )
    b_cat = jnp.concatenate(
        [b_w, b_h, b_l, b_d, jnp.zeros((1,), jnp.float32)], axis=0
    ).reshape(N_PAD, 1)

    grid = (B, T // BLOCK_T)
    ow, oh, ol, od = pl.pallas_call(
        _fused_heads_kernel,
        grid=grid,
        in_specs=[
            pl.BlockSpec((1, BLOCK_T, D), lambda b, j: (b, j, 0)),
            pl.BlockSpec((N_PAD, D), lambda b, j: (0, 0)),
            pl.BlockSpec((N_PAD, 1), lambda b, j: (0, 0)),
        ],
        out_specs=[
            pl.BlockSpec((1, 16, BLOCK_T), lambda b, j: (b, 0, j)),
            pl.BlockSpec((1, 16, BLOCK_T), lambda b, j: (b, 0, j)),
            pl.BlockSpec((1, 16, BLOCK_T), lambda b, j: (b, 0, j)),
            pl.BlockSpec((1, 15, BLOCK_T), lambda b, j: (b, 0, j)),
        ],
        out_shape=[
            jax.ShapeDtypeStruct((B, 16, T), jnp.float32),
            jax.ShapeDtypeStruct((B, 16, T), jnp.float32),
            jax.ShapeDtypeStruct((B, 16, T), jnp.float32),
            jax.ShapeDtypeStruct((B, 15, T), jnp.float32),
        ],
        compiler_params=pltpu.CompilerParams(
            dimension_semantics=("parallel", "parallel"),
        ),
    )(conditioning, W_cat, b_cat)

    return (
        ow.transpose(0, 2, 1),
        oh.transpose(0, 2, 1),
        od.transpose(0, 2, 1),
        ol.transpose(0, 2, 1),
    )
